# baseline (speedup 1.0000x reference)
import functools
import math

import jax
import jax.numpy as jnp
from jax import lax
from jax.experimental import pallas as pl
from jax.experimental.pallas import tpu as pltpu
from jax.experimental.pallas import tpu_sc as plsc

N = 32768
REP = 8
W = 16
CH = N // W
NV = CH // 16
MIN_S = -0.1
MAX_S = 1.1
EPS = 1e-06
MAGIC = 0.8
BETA = 0.5
_X = (0.0 - MIN_S) / (MAX_S - MIN_S)
LOGITS_BETA = float((math.log(_X) - math.log(1.0 - _X)) * BETA)

SCRATCH = dict(
    la_v=pltpu.VMEM((CH,), jnp.float32),
    z_v=pltpu.VMEM((CH,), jnp.float32),
    bits_v=pltpu.VMEM((CH,), jnp.int32),
    d_v=pltpu.VMEM((CH,), jnp.int32),
    hist_v=pltpu.VMEM((256,), jnp.int32),
    allhist_v=pltpu.VMEM((W * 256,), jnp.int32),
    row_f=pltpu.VMEM((16,), jnp.float32),
    row_i=pltpu.VMEM((16,), jnp.int32),
    allrow_f=pltpu.VMEM((W * 16,), jnp.float32),
    allrow_i=pltpu.VMEM((W * 16,), jnp.int32),
    out_v=pltpu.VMEM((REP * CH,), jnp.float32),
    sh_hist=pltpu.VMEM_SHARED((W * 256,), jnp.int32),
    sh_f=pltpu.VMEM_SHARED((W * 16,), jnp.float32),
    sh_i=pltpu.VMEM_SHARED((W * 16,), jnp.int32),
    sem=pltpu.SemaphoreType.DMA,
)


def _mask_body(la_hbm, out_hbm, la_v, z_v, bits_v, d_v, hist_v, allhist_v,
               row_f, row_i, allrow_f, allrow_i, out_v, sh_hist, sh_f, sh_i,
               sem):
    wid = lax.axis_index("s")
    base = wid * CH
    ones_i = jnp.full((16,), 1, jnp.int32)

    pltpu.sync_copy(la_hbm.at[pl.ds(base, CH)], la_v)

    def phase_a(j, acc):
        x = la_v[pl.ds(j * 16, 16)]
        ell = 1.0 / (1.0 + jnp.exp(-(x - LOGITS_BETA)))
        ell = jnp.clip(ell, jnp.float32(EPS), jnp.float32(1.0 - EPS))
        z = 1.0 / (1.0 + jnp.exp(-(x / jnp.float32(BETA) * jnp.float32(MAGIC))))
        z_v[pl.ds(j * 16, 16)] = z
        bits_v[pl.ds(j * 16, 16)] = plsc.bitcast(z, jnp.int32)
        return acc + ell

    acc = lax.fori_loop(0, NV, phase_a, jnp.zeros((16,), jnp.float32),
                        unroll=8)
    lsum = jnp.sum(acc)

    row_f[...] = jnp.full((16,), lsum, jnp.float32)
    pltpu.sync_copy(row_f, sh_f.at[pl.ds(wid * 16, 16)])
    plsc.subcore_barrier()
    pltpu.sync_copy(sh_f, allrow_f)
    tot = jnp.zeros((16,), jnp.float32)
    for w in range(W):
        tot = tot + allrow_f[pl.ds(w * 16, 16)]
    lc = jnp.max(tot)

    y = jnp.float32(N) - lc
    n = y.astype(jnp.int32)
    frac = y - n.astype(jnp.float32)
    k = n + jnp.where(frac > 0.5, 1, 0) + jnp.where(
        (frac == 0.5) & (n % 2 == 1), 1, 0)
    kk = jnp.clip(k, 1, N)

    @pl.when(k == 0)
    def _fast():
        copies = [
            pltpu.async_copy(z_v, out_hbm.at[pl.ds(r * N + base, CH)], sem)
            for r in range(REP)
        ]
        for cp in copies:
            cp.wait()

    @pl.when(k > 0)
    def _general():
        _masked_paths(k, kk, wid, base, ones_i, out_hbm, z_v, bits_v, d_v,
                      hist_v, allhist_v, row_i, allrow_i, out_v, sh_hist,
                      sh_i, sem)


def _masked_paths(k, kk, wid, base, ones_i, out_hbm, z_v, bits_v, d_v, hist_v,
                  allhist_v, row_i, allrow_i, out_v, sh_hist, sh_i, sem):
    prefix = jnp.int32(0)
    below = jnp.int32(0)
    rem = kk - 1
    bcount = jnp.int32(0)
    for p in range(4):
        shift = 24 - 8 * p
        for t in range(16):
            hist_v[pl.ds(t * 16, 16)] = jnp.zeros((16,), jnp.int32)

        if p == 0:
            def build(j, carry):
                b = bits_v[pl.ds(j * 16, 16)]
                idx = lax.shift_right_logical(b, shift) & 255
                plsc.addupdate_scatter(hist_v, [idx], ones_i)
                return carry
        else:
            pshift = lax.shift_right_logical(prefix, shift + 8)

            def build(j, carry):
                b = bits_v[pl.ds(j * 16, 16)]
                match = lax.shift_right_logical(b, shift + 8) == pshift
                idx = lax.shift_right_logical(b, shift) & 255
                plsc.addupdate_scatter(hist_v, [idx], ones_i, mask=match)
                return carry

        lax.fori_loop(0, NV, build, jnp.int32(0))

        pltpu.sync_copy(hist_v, sh_hist.at[pl.ds(wid * 256, 256)])
        plsc.subcore_barrier()
        pltpu.sync_copy(sh_hist, allhist_v)
        plsc.subcore_barrier()

        carry = jnp.int32(0)
        q16 = jnp.int32(0)
        cbefore = jnp.int32(0)
        cat = jnp.int32(0x7FFFFFFF)
        for t in range(16):
            h = jnp.zeros((16,), jnp.int32)
            for w in range(W):
                h = h + allhist_v[pl.ds(w * 256 + t * 16, 16)]
            c = plsc.cumsum(h) + carry
            le = c <= rem
            q16 = q16 + jnp.sum(jnp.where(le, 1, 0))
            cbefore = jnp.maximum(cbefore, jnp.max(jnp.where(le, c, 0)))
            cat = jnp.minimum(cat, jnp.min(jnp.where(le, jnp.int32(0x7FFFFFFF), c)))
            carry = jnp.max(c)
        q = q16
        prefix = prefix | lax.shift_left(q, shift)
        below = below + cbefore
        rem = rem - cbefore
        bcount = cat - cbefore

    vstar = prefix
    astar = below
    bstar = bcount
    tcut = 8 * (k - astar)
    vstar_vec = jnp.full((16,), vstar, jnp.int32)

    straddle = (tcut > 0) & (tcut < 8 * bstar)

    @pl.when(jnp.logical_not(straddle))
    def _uniform():
        zero_eq = jnp.full((16,), tcut >= 8 * bstar, jnp.bool_)

        def emit(j, carry):
            b = bits_v[pl.ds(j * 16, 16)]
            z = z_v[pl.ds(j * 16, 16)]
            zero = (b < vstar_vec) | ((b == vstar_vec) & zero_eq)
            out_v[pl.ds(j * 16, 16)] = jnp.where(zero, jnp.float32(0.0), z)
            return carry

        lax.fori_loop(0, NV, emit, jnp.int32(0))
        copies = [
            pltpu.async_copy(out_v.at[pl.ds(0, CH)],
                             out_hbm.at[pl.ds(r * N + base, CH)], sem)
            for r in range(REP)
        ]
        for cp in copies:
            cp.wait()

    @pl.when(straddle)
    def _tie_split():
        def tie_rank(j, carry):
            b = bits_v[pl.ds(j * 16, 16)]
            eq = jnp.where(b == vstar_vec, 1, 0)
            c = plsc.cumsum(eq)
            d_v[pl.ds(j * 16, 16)] = c - eq + carry
            return carry + jnp.max(c)

        eq_tot = lax.fori_loop(0, NV, tie_rank, jnp.int32(0))
        row_i[...] = jnp.full((16,), eq_tot, jnp.int32)
        pltpu.sync_copy(row_i, sh_i.at[pl.ds(wid * 16, 16)])
        plsc.subcore_barrier()
        pltpu.sync_copy(sh_i, allrow_i)
        plsc.subcore_barrier()
        eq_before = jnp.int32(0)
        for w in range(W):
            eq_before = eq_before + jnp.where(
                jnp.int32(w) < wid, jnp.max(allrow_i[pl.ds(w * 16, 16)]), 0)

        ebvec = jnp.full((16,), eq_before, jnp.int32)
        tvec = jnp.full((16,), tcut, jnp.int32)
        for r in range(REP):
            rb = jnp.full((16,), jnp.int32(r) * bstar, jnp.int32)

            def emit(j, carry, r=r, rb=rb):
                b = bits_v[pl.ds(j * 16, 16)]
                z = z_v[pl.ds(j * 16, 16)]
                d = d_v[pl.ds(j * 16, 16)] + ebvec
                zero = (b < vstar_vec) | ((b == vstar_vec) & (rb + d < tvec))
                out_v[pl.ds(r * CH + j * 16, 16)] = jnp.where(
                    zero, jnp.float32(0.0), z)
                return carry

            lax.fori_loop(0, NV, emit, jnp.int32(0))

        copies = [
            pltpu.async_copy(out_v.at[pl.ds(r * CH, CH)],
                             out_hbm.at[pl.ds(r * N + base, CH)], sem)
            for r in range(REP)
        ]
        for cp in copies:
            cp.wait()


def _make(interpret=False):
    mesh = plsc.VectorSubcoreMesh(
        core_axis_name="c", subcore_axis_name="s", num_cores=1, num_subcores=W)
    return pl.kernel(
        _mask_body,
        out_type=jax.ShapeDtypeStruct((N * REP,), jnp.float32),
        mesh=mesh,
        compiler_params=pltpu.CompilerParams(needs_layout_passes=False),
        interpret=interpret,
        scratch_types=SCRATCH,
    )


def kernel(log_alpha):
    return _make()(log_alpha)


# speedup vs baseline: 1.0926x; 1.0926x over previous
import functools
import math

import jax
import jax.numpy as jnp
from jax import lax
from jax.experimental import pallas as pl
from jax.experimental.pallas import tpu as pltpu
from jax.experimental.pallas import tpu_sc as plsc

N = 32768
REP = 8
W = 16
CH = N // W
NV = CH // 16
MIN_S = -0.1
MAX_S = 1.1
EPS = 1e-06
MAGIC = 0.8
BETA = 0.5
_X = (0.0 - MIN_S) / (MAX_S - MIN_S)
LOGITS_BETA = float((math.log(_X) - math.log(1.0 - _X)) * BETA)

SCRATCH = dict(
    la_v=pltpu.VMEM((CH,), jnp.float32),
    z_v=pltpu.VMEM((CH,), jnp.float32),
    bits_v=pltpu.VMEM((CH,), jnp.int32),
    d_v=pltpu.VMEM((CH,), jnp.int32),
    hist_v=pltpu.VMEM((256,), jnp.int32),
    allhist_v=pltpu.VMEM((W * 256,), jnp.int32),
    row_f=pltpu.VMEM((16,), jnp.float32),
    row_i=pltpu.VMEM((16,), jnp.int32),
    allrow_f=pltpu.VMEM((W * 16,), jnp.float32),
    allrow_i=pltpu.VMEM((W * 16,), jnp.int32),
    out_v=pltpu.VMEM((REP * CH,), jnp.float32),
    sh_hist=pltpu.VMEM_SHARED((W * 256,), jnp.int32),
    sh_f=pltpu.VMEM_SHARED((W * 16,), jnp.float32),
    sh_i=pltpu.VMEM_SHARED((W * 16,), jnp.int32),
    sem=pltpu.SemaphoreType.DMA,
)


def _mask_body(la_hbm, out_hbm, la_v, z_v, bits_v, d_v, hist_v, allhist_v,
               row_f, row_i, allrow_f, allrow_i, out_v, sh_hist, sh_f, sh_i,
               sem):
    wid = lax.axis_index("s")
    base = wid * CH
    ones_i = jnp.full((16,), 1, jnp.int32)

    pltpu.sync_copy(la_hbm.at[pl.ds(base, CH)], la_v)

    def phase_a(j, acc):
        x = la_v[pl.ds(j * 16, 16)]
        ell = 1.0 / (1.0 + jnp.exp(-(x - LOGITS_BETA)))
        ell = jnp.clip(ell, jnp.float32(EPS), jnp.float32(1.0 - EPS))
        z = 1.0 / (1.0 + jnp.exp(-(x / jnp.float32(BETA) * jnp.float32(MAGIC))))
        z_v[pl.ds(j * 16, 16)] = z
        bits_v[pl.ds(j * 16, 16)] = plsc.bitcast(z, jnp.int32)
        return acc + ell

    acc = lax.fori_loop(0, NV, phase_a, jnp.zeros((16,), jnp.float32))
    lsum = jnp.sum(acc)

    row_f[...] = jnp.full((16,), lsum, jnp.float32)
    pltpu.sync_copy(row_f, sh_f.at[pl.ds(wid * 16, 16)])
    plsc.subcore_barrier()
    pltpu.sync_copy(sh_f, allrow_f)
    tot = jnp.zeros((16,), jnp.float32)
    for w in range(W):
        tot = tot + allrow_f[pl.ds(w * 16, 16)]
    lc = jnp.max(tot)

    y = jnp.float32(N) - lc
    n = y.astype(jnp.int32)
    frac = y - n.astype(jnp.float32)
    k = n + jnp.where(frac > 0.5, 1, 0) + jnp.where(
        (frac == 0.5) & (n % 2 == 1), 1, 0)
    kk = jnp.clip(k, 1, N)

    @pl.when(k == 0)
    def _fast():
        copies = [
            pltpu.async_copy(z_v, out_hbm.at[pl.ds(r * N + base, CH)], sem)
            for r in range(REP)
        ]
        for cp in copies:
            cp.wait()

    @pl.when(k > 0)
    def _general():
        _masked_paths(k, kk, wid, base, ones_i, out_hbm, z_v, bits_v, d_v,
                      hist_v, allhist_v, row_i, allrow_i, out_v, sh_hist,
                      sh_i, sem)


def _masked_paths(k, kk, wid, base, ones_i, out_hbm, z_v, bits_v, d_v, hist_v,
                  allhist_v, row_i, allrow_i, out_v, sh_hist, sh_i, sem):
    prefix = jnp.int32(0)
    below = jnp.int32(0)
    rem = kk - 1
    bcount = jnp.int32(0)
    for p in range(4):
        shift = 24 - 8 * p
        for t in range(16):
            hist_v[pl.ds(t * 16, 16)] = jnp.zeros((16,), jnp.int32)

        if p == 0:
            def build(j, carry):
                b = bits_v[pl.ds(j * 16, 16)]
                idx = lax.shift_right_logical(b, shift) & 255
                plsc.addupdate_scatter(hist_v, [idx], ones_i)
                return carry
        else:
            pshift = lax.shift_right_logical(prefix, shift + 8)

            def build(j, carry):
                b = bits_v[pl.ds(j * 16, 16)]
                match = lax.shift_right_logical(b, shift + 8) == pshift
                idx = lax.shift_right_logical(b, shift) & 255
                plsc.addupdate_scatter(hist_v, [idx], ones_i, mask=match)
                return carry

        lax.fori_loop(0, NV, build, jnp.int32(0))

        pltpu.sync_copy(hist_v, sh_hist.at[pl.ds(wid * 256, 256)])
        plsc.subcore_barrier()
        pltpu.sync_copy(sh_hist, allhist_v)
        plsc.subcore_barrier()

        carry = jnp.int32(0)
        q16 = jnp.int32(0)
        cbefore = jnp.int32(0)
        cat = jnp.int32(0x7FFFFFFF)
        for t in range(16):
            h = jnp.zeros((16,), jnp.int32)
            for w in range(W):
                h = h + allhist_v[pl.ds(w * 256 + t * 16, 16)]
            c = plsc.cumsum(h) + carry
            le = c <= rem
            q16 = q16 + jnp.sum(jnp.where(le, 1, 0))
            cbefore = jnp.maximum(cbefore, jnp.max(jnp.where(le, c, 0)))
            cat = jnp.minimum(cat, jnp.min(jnp.where(le, jnp.int32(0x7FFFFFFF), c)))
            carry = jnp.max(c)
        q = q16
        prefix = prefix | lax.shift_left(q, shift)
        below = below + cbefore
        rem = rem - cbefore
        bcount = cat - cbefore

    vstar = prefix
    astar = below
    bstar = bcount
    tcut = 8 * (k - astar)
    vstar_vec = jnp.full((16,), vstar, jnp.int32)

    straddle = (tcut > 0) & (tcut < 8 * bstar)

    @pl.when(jnp.logical_not(straddle))
    def _uniform():
        zero_eq = jnp.full((16,), tcut >= 8 * bstar, jnp.bool_)

        def emit(j, carry):
            b = bits_v[pl.ds(j * 16, 16)]
            z = z_v[pl.ds(j * 16, 16)]
            zero = (b < vstar_vec) | ((b == vstar_vec) & zero_eq)
            out_v[pl.ds(j * 16, 16)] = jnp.where(zero, jnp.float32(0.0), z)
            return carry

        lax.fori_loop(0, NV, emit, jnp.int32(0))
        copies = [
            pltpu.async_copy(out_v.at[pl.ds(0, CH)],
                             out_hbm.at[pl.ds(r * N + base, CH)], sem)
            for r in range(REP)
        ]
        for cp in copies:
            cp.wait()

    @pl.when(straddle)
    def _tie_split():
        def tie_rank(j, carry):
            b = bits_v[pl.ds(j * 16, 16)]
            eq = jnp.where(b == vstar_vec, 1, 0)
            c = plsc.cumsum(eq)
            d_v[pl.ds(j * 16, 16)] = c - eq + carry
            return carry + jnp.max(c)

        eq_tot = lax.fori_loop(0, NV, tie_rank, jnp.int32(0))
        row_i[...] = jnp.full((16,), eq_tot, jnp.int32)
        pltpu.sync_copy(row_i, sh_i.at[pl.ds(wid * 16, 16)])
        plsc.subcore_barrier()
        pltpu.sync_copy(sh_i, allrow_i)
        plsc.subcore_barrier()
        eq_before = jnp.int32(0)
        for w in range(W):
            eq_before = eq_before + jnp.where(
                jnp.int32(w) < wid, jnp.max(allrow_i[pl.ds(w * 16, 16)]), 0)

        ebvec = jnp.full((16,), eq_before, jnp.int32)
        tvec = jnp.full((16,), tcut, jnp.int32)
        for r in range(REP):
            rb = jnp.full((16,), jnp.int32(r) * bstar, jnp.int32)

            def emit(j, carry, r=r, rb=rb):
                b = bits_v[pl.ds(j * 16, 16)]
                z = z_v[pl.ds(j * 16, 16)]
                d = d_v[pl.ds(j * 16, 16)] + ebvec
                zero = (b < vstar_vec) | ((b == vstar_vec) & (rb + d < tvec))
                out_v[pl.ds(r * CH + j * 16, 16)] = jnp.where(
                    zero, jnp.float32(0.0), z)
                return carry

            lax.fori_loop(0, NV, emit, jnp.int32(0))

        copies = [
            pltpu.async_copy(out_v.at[pl.ds(r * CH, CH)],
                             out_hbm.at[pl.ds(r * N + base, CH)], sem)
            for r in range(REP)
        ]
        for cp in copies:
            cp.wait()


def _make(interpret=False):
    mesh = plsc.VectorSubcoreMesh(
        core_axis_name="c", subcore_axis_name="s", num_cores=1, num_subcores=W)
    return pl.kernel(
        _mask_body,
        out_type=jax.ShapeDtypeStruct((N * REP,), jnp.float32),
        mesh=mesh,
        compiler_params=pltpu.CompilerParams(
            needs_layout_passes=False, skip_device_barrier=True),
        interpret=interpret,
        scratch_types=SCRATCH,
    )


def kernel(log_alpha):
    return _make()(log_alpha)


# speedup vs baseline: 1.1052x; 1.0115x over previous
import functools
import math

import jax
import jax.numpy as jnp
from jax import lax
from jax.experimental import pallas as pl
from jax.experimental.pallas import tpu as pltpu
from jax.experimental.pallas import tpu_sc as plsc

N = 32768
REP = 8
W = 16
CH = N // W
NV = CH // 16
MIN_S = -0.1
MAX_S = 1.1
EPS = 1e-06
MAGIC = 0.8
BETA = 0.5
_X = (0.0 - MIN_S) / (MAX_S - MIN_S)
LOGITS_BETA = float((math.log(_X) - math.log(1.0 - _X)) * BETA)

SCRATCH = dict(
    la_v=pltpu.VMEM((CH,), jnp.float32),
    z_v=pltpu.VMEM((CH,), jnp.float32),
    bits_v=pltpu.VMEM((CH,), jnp.int32),
    d_v=pltpu.VMEM((CH,), jnp.int32),
    hist_v=pltpu.VMEM((256,), jnp.int32),
    allhist_v=pltpu.VMEM((W * 256,), jnp.int32),
    row_f=pltpu.VMEM((16,), jnp.float32),
    row_i=pltpu.VMEM((16,), jnp.int32),
    allrow_f=pltpu.VMEM((W * 16,), jnp.float32),
    allrow_i=pltpu.VMEM((W * 16,), jnp.int32),
    out_v=pltpu.VMEM((REP * CH,), jnp.float32),
    sh_hist=pltpu.VMEM_SHARED((W * 256,), jnp.int32),
    sh_f=pltpu.VMEM_SHARED((W * 16,), jnp.float32),
    sh_i=pltpu.VMEM_SHARED((W * 16,), jnp.int32),
    sem=pltpu.SemaphoreType.DMA,
)


def _mask_body(la_hbm, out_hbm, la_v, z_v, bits_v, d_v, hist_v, allhist_v,
               row_f, row_i, allrow_f, allrow_i, out_v, sh_hist, sh_f, sh_i,
               sem):
    wid = lax.axis_index("s")
    base = wid * CH
    ones_i = jnp.full((16,), 1, jnp.int32)

    pltpu.sync_copy(la_hbm.at[pl.ds(base, CH)], la_v)

    def phase_a(j, acc):
        x = la_v[pl.ds(j * 16, 16)]
        ell = 1.0 / (1.0 + jnp.exp(-(x - LOGITS_BETA)))
        ell = jnp.clip(ell, jnp.float32(EPS), jnp.float32(1.0 - EPS))
        z = 1.0 / (1.0 + jnp.exp(-(x / jnp.float32(BETA) * jnp.float32(MAGIC))))
        z_v[pl.ds(j * 16, 16)] = z
        bits_v[pl.ds(j * 16, 16)] = plsc.bitcast(z, jnp.int32)
        return acc + ell

    acc = lax.fori_loop(0, NV, phase_a, jnp.zeros((16,), jnp.float32))
    lsum = jnp.sum(acc)

    zcopies = [
        pltpu.async_copy(z_v, out_hbm.at[pl.ds(r * N + base, CH)], sem)
        for r in range(REP)
    ]

    row_f[...] = jnp.full((16,), lsum, jnp.float32)
    pltpu.sync_copy(row_f, sh_f.at[pl.ds(wid * 16, 16)])
    plsc.subcore_barrier()
    pltpu.sync_copy(sh_f, allrow_f)
    tot = jnp.zeros((16,), jnp.float32)
    for w in range(W):
        tot = tot + allrow_f[pl.ds(w * 16, 16)]
    lc = jnp.max(tot)

    y = jnp.float32(N) - lc
    n = y.astype(jnp.int32)
    frac = y - n.astype(jnp.float32)
    k = n + jnp.where(frac > 0.5, 1, 0) + jnp.where(
        (frac == 0.5) & (n % 2 == 1), 1, 0)
    kk = jnp.clip(k, 1, N)

    for cp in zcopies:
        cp.wait()

    @pl.when(k > 0)
    def _general():
        _masked_paths(k, kk, wid, base, ones_i, out_hbm, z_v, bits_v, d_v,
                      hist_v, allhist_v, row_i, allrow_i, out_v, sh_hist,
                      sh_i, sem)


def _masked_paths(k, kk, wid, base, ones_i, out_hbm, z_v, bits_v, d_v, hist_v,
                  allhist_v, row_i, allrow_i, out_v, sh_hist, sh_i, sem):
    prefix = jnp.int32(0)
    below = jnp.int32(0)
    rem = kk - 1
    bcount = jnp.int32(0)
    for p in range(4):
        shift = 24 - 8 * p
        for t in range(16):
            hist_v[pl.ds(t * 16, 16)] = jnp.zeros((16,), jnp.int32)

        if p == 0:
            def build(j, carry):
                b = bits_v[pl.ds(j * 16, 16)]
                idx = lax.shift_right_logical(b, shift) & 255
                plsc.addupdate_scatter(hist_v, [idx], ones_i)
                return carry
        else:
            pshift = lax.shift_right_logical(prefix, shift + 8)

            def build(j, carry):
                b = bits_v[pl.ds(j * 16, 16)]
                match = lax.shift_right_logical(b, shift + 8) == pshift
                idx = lax.shift_right_logical(b, shift) & 255
                plsc.addupdate_scatter(hist_v, [idx], ones_i, mask=match)
                return carry

        lax.fori_loop(0, NV, build, jnp.int32(0))

        pltpu.sync_copy(hist_v, sh_hist.at[pl.ds(wid * 256, 256)])
        plsc.subcore_barrier()
        pltpu.sync_copy(sh_hist, allhist_v)
        plsc.subcore_barrier()

        carry = jnp.int32(0)
        q16 = jnp.int32(0)
        cbefore = jnp.int32(0)
        cat = jnp.int32(0x7FFFFFFF)
        for t in range(16):
            h = jnp.zeros((16,), jnp.int32)
            for w in range(W):
                h = h + allhist_v[pl.ds(w * 256 + t * 16, 16)]
            c = plsc.cumsum(h) + carry
            le = c <= rem
            q16 = q16 + jnp.sum(jnp.where(le, 1, 0))
            cbefore = jnp.maximum(cbefore, jnp.max(jnp.where(le, c, 0)))
            cat = jnp.minimum(cat, jnp.min(jnp.where(le, jnp.int32(0x7FFFFFFF), c)))
            carry = jnp.max(c)
        q = q16
        prefix = prefix | lax.shift_left(q, shift)
        below = below + cbefore
        rem = rem - cbefore
        bcount = cat - cbefore

    vstar = prefix
    astar = below
    bstar = bcount
    tcut = 8 * (k - astar)
    vstar_vec = jnp.full((16,), vstar, jnp.int32)

    straddle = (tcut > 0) & (tcut < 8 * bstar)

    @pl.when(jnp.logical_not(straddle))
    def _uniform():
        zero_eq = jnp.full((16,), tcut >= 8 * bstar, jnp.bool_)

        def emit(j, carry):
            b = bits_v[pl.ds(j * 16, 16)]
            z = z_v[pl.ds(j * 16, 16)]
            zero = (b < vstar_vec) | ((b == vstar_vec) & zero_eq)
            out_v[pl.ds(j * 16, 16)] = jnp.where(zero, jnp.float32(0.0), z)
            return carry

        lax.fori_loop(0, NV, emit, jnp.int32(0))
        copies = [
            pltpu.async_copy(out_v.at[pl.ds(0, CH)],
                             out_hbm.at[pl.ds(r * N + base, CH)], sem)
            for r in range(REP)
        ]
        for cp in copies:
            cp.wait()

    @pl.when(straddle)
    def _tie_split():
        def tie_rank(j, carry):
            b = bits_v[pl.ds(j * 16, 16)]
            eq = jnp.where(b == vstar_vec, 1, 0)
            c = plsc.cumsum(eq)
            d_v[pl.ds(j * 16, 16)] = c - eq + carry
            return carry + jnp.max(c)

        eq_tot = lax.fori_loop(0, NV, tie_rank, jnp.int32(0))
        row_i[...] = jnp.full((16,), eq_tot, jnp.int32)
        pltpu.sync_copy(row_i, sh_i.at[pl.ds(wid * 16, 16)])
        plsc.subcore_barrier()
        pltpu.sync_copy(sh_i, allrow_i)
        plsc.subcore_barrier()
        eq_before = jnp.int32(0)
        for w in range(W):
            eq_before = eq_before + jnp.where(
                jnp.int32(w) < wid, jnp.max(allrow_i[pl.ds(w * 16, 16)]), 0)

        ebvec = jnp.full((16,), eq_before, jnp.int32)
        tvec = jnp.full((16,), tcut, jnp.int32)
        for r in range(REP):
            rb = jnp.full((16,), jnp.int32(r) * bstar, jnp.int32)

            def emit(j, carry, r=r, rb=rb):
                b = bits_v[pl.ds(j * 16, 16)]
                z = z_v[pl.ds(j * 16, 16)]
                d = d_v[pl.ds(j * 16, 16)] + ebvec
                zero = (b < vstar_vec) | ((b == vstar_vec) & (rb + d < tvec))
                out_v[pl.ds(r * CH + j * 16, 16)] = jnp.where(
                    zero, jnp.float32(0.0), z)
                return carry

            lax.fori_loop(0, NV, emit, jnp.int32(0))

        copies = [
            pltpu.async_copy(out_v.at[pl.ds(r * CH, CH)],
                             out_hbm.at[pl.ds(r * N + base, CH)], sem)
            for r in range(REP)
        ]
        for cp in copies:
            cp.wait()


def _make(interpret=False):
    mesh = plsc.VectorSubcoreMesh(
        core_axis_name="c", subcore_axis_name="s", num_cores=1, num_subcores=W)
    return pl.kernel(
        _mask_body,
        out_type=jax.ShapeDtypeStruct((N * REP,), jnp.float32),
        mesh=mesh,
        compiler_params=pltpu.CompilerParams(
            needs_layout_passes=False, skip_device_barrier=True),
        interpret=interpret,
        scratch_types=SCRATCH,
    )


def kernel(log_alpha):
    return _make()(log_alpha)
